# Initial kernel scaffold; baseline (speedup 1.0000x reference)
import functools

import jax
import jax.numpy as jnp
from jax import lax
from jax.experimental import pallas as pl
from jax.experimental.pallas import tpu as pltpu
from jax.experimental.pallas import tpu_sc as plsc

N = 10000
B = 32
K = 16
DF = 153
DL = 256
H = 128

NPAD = 10240
R = 256
CCH = 512
NBLK = NPAD // R
NCH = NPAD // CCH
DPAD = 256

BIG = jnp.float32(1e9)
FINF = jnp.float32(3e38)

NC = 2
NS = 16
NW = NC * NS
NODES_PW = NPAD // NW
GCH = 8
NGB = 4
STEP = GCH * NGB
OUTER = NODES_PW // STEP


def _lang_body(lang_ref, wl1_ref, bl1_ref, bng_ref, bnb_ref, wl2_ref, bl2_ref,
               out_ref):
    h = jnp.dot(lang_ref[...], wl1_ref[...],
                preferred_element_type=jnp.float32) + bl1_ref[...][None, :]
    mu = jnp.mean(h, axis=0, keepdims=True)
    var = jnp.mean((h - mu) ** 2, axis=0, keepdims=True)
    h = (h - mu) / jnp.sqrt(var + 1e-5) * bng_ref[...][None, :] \
        + bnb_ref[...][None, :]
    h = jnp.maximum(h, 0.0)
    out_ref[...] = jnp.dot(h, wl2_ref[...],
                           preferred_element_type=jnp.float32) \
        + bl2_ref[...][None, :]


def _pre_body(f_ref, wa_ref, wc_ref, be_ref, a_ref, c_ref):
    f = f_ref[...]
    a_ref[...] = jnp.dot(f, wa_ref[...],
                         preferred_element_type=jnp.float32) \
        + be_ref[...][None, :]
    c_ref[...] = jnp.dot(f, wc_ref[...], preferred_element_type=jnp.float32)


def _knn_body(bounds_ref, p_ref, pt3_ref, out_ref):
    blk = pl.program_id(0)
    r0 = blk * R
    row = p_ref[...]
    xr = row[:, 0:1]
    yr = row[:, 1:2]
    zr = row[:, 2:3]
    sqr = row[:, 3:4]
    bir = row[:, 4:5]
    rowid = r0 + lax.broadcasted_iota(jnp.int32, (R, 1), 0)
    kc0 = bounds_ref[blk, 0]
    kc1 = bounds_ref[blk, 1]

    init_v = jnp.full((R, K), FINF, jnp.float32)
    init_i = jnp.broadcast_to(
        lax.broadcasted_iota(jnp.int32, (1, K), 1) + NPAD, (R, K))

    def chunk_body(kc, carry):
        cv, ci = carry
        c0 = kc * CCH
        ch = pt3_ref[kc]
        xc = ch[0:1, :]
        yc = ch[1:2, :]
        zc = ch[2:3, :]
        sqc = ch[3:4, :]
        bic = ch[4:5, :]
        dot = xr * xc + yr * yc + zr * zc
        d2 = sqr + sqc - 2.0 * dot
        colid = c0 + lax.broadcasted_iota(jnp.int32, (1, CCH), 1)
        d2 = d2 + jnp.where(bir != bic, BIG, 0.0) \
            + jnp.where(rowid == colid, BIG, 0.0)
        cand_v = jnp.concatenate([cv, d2], axis=1)
        cand_i = jnp.concatenate(
            [ci, jnp.broadcast_to(colid, (R, CCH))], axis=1)
        vs = []
        isel = []
        for _ in range(K):
            m = jnp.min(cand_v, axis=1, keepdims=True)
            si = jnp.min(jnp.where(cand_v == m, cand_i, jnp.int32(2**31 - 1)),
                         axis=1, keepdims=True)
            vs.append(m)
            isel.append(si)
            cand_v = jnp.where(cand_i == si, FINF, cand_v)
        return jnp.concatenate(vs, axis=1), jnp.concatenate(isel, axis=1)

    _, idx = lax.fori_loop(kc0, kc1, chunk_body, (init_v, init_i))
    out_ref[...] = jnp.clip(idx, 0, NPAD - 1)


def _vis_body(a_ref, g_ref, p_ref, lang_ref, wv1_ref, bv1_ref, lng_ref,
              lnb_ref, wv2_ref, bv2_ref, out_ref):
    node = jnp.maximum(a_ref[...] + g_ref[...], 0.0)
    v = jnp.dot(node, wv1_ref[...],
                preferred_element_type=jnp.float32) + bv1_ref[...][None, :]
    mu = jnp.mean(v, axis=1, keepdims=True)
    var = jnp.mean((v - mu) ** 2, axis=1, keepdims=True)
    v = (v - mu) / jnp.sqrt(var + 1e-5) * lng_ref[...][None, :] \
        + lnb_ref[...][None, :]
    v = jnp.maximum(v, 0.0)
    v = jnp.dot(v, wv2_ref[...],
                preferred_element_type=jnp.float32) + bv2_ref[...][None, :]
    bi = p_ref[...][:, 4:5]
    ioti = lax.broadcasted_iota(jnp.int32, (R, B), 1).astype(jnp.float32)
    onehot = (bi == ioti).astype(jnp.float32)
    lf = jnp.dot(onehot, lang_ref[...], preferred_element_type=jnp.float32)
    num = jnp.sum(v * lf, axis=1, keepdims=True)
    den = jnp.sqrt(jnp.sum(v * v, axis=1, keepdims=True)) \
        * jnp.sqrt(jnp.sum(lf * lf, axis=1, keepdims=True)) + 1e-8
    out_ref[...] = num / den


def _sc_gather_max(idx_flat, ctab):
    mesh = plsc.VectorSubcoreMesh(core_axis_name="c", subcore_axis_name="s",
                                  num_cores=NC, num_subcores=NS)

    @functools.partial(
        pl.kernel,
        out_type=jax.ShapeDtypeStruct((NPAD, H), jnp.float32),
        mesh=mesh,
        scratch_types=(
            [pltpu.VMEM((GCH * K,), jnp.int32) for _ in range(NGB)]
            + [pltpu.VMEM((GCH * K, H), jnp.float32) for _ in range(NGB)]
            + [pltpu.VMEM((STEP, H), jnp.float32), pltpu.SemaphoreType.DMA]
        ),
    )
    def kfn(idx_hbm, c_hbm, out_hbm, i0, i1, i2, i3, r0, r1, r2, r3, outv,
            sem):
        ivs = (i0, i1, i2, i3)
        rvs = (r0, r1, r2, r3)
        wid = lax.axis_index("s") * NC + lax.axis_index("c")
        base = wid * NODES_PW

        def outer(t, carry):
            node0 = base + t * STEP
            for j in range(NGB):
                pltpu.sync_copy(
                    idx_hbm.at[pl.ds((node0 + j * GCH) * K, GCH * K)], ivs[j])
            cps = [pltpu.async_copy(c_hbm.at[ivs[j]], rvs[j], sem)
                   for j in range(NGB)]
            for cp in cps:
                cp.wait()
            for j in range(NGB):
                rv = rvs[j]

                def nbody(n, c2, rv=rv, j=j):
                    rbase = n * K
                    for dg in range(H // 16):
                        acc = rv[rbase, pl.ds(dg * 16, 16)]
                        for k in range(1, K):
                            acc = jnp.maximum(
                                acc, rv[rbase + k, pl.ds(dg * 16, 16)])
                        outv[j * GCH + n, pl.ds(dg * 16, 16)] = acc
                    return c2

                lax.fori_loop(0, GCH, nbody, 0)
            pltpu.sync_copy(outv, out_hbm.at[pl.ds(node0, STEP)])
            return carry

        lax.fori_loop(0, OUTER, outer, 0)

    return kfn(idx_flat, ctab)


def kernel(feats153, support_xyz, batch_index, filtered_index, lang_rel_feats,
           W_l1, b_l1, bn_g, bn_b, W_l2, b_l2, W_e, b_e, W_v1, b_v1, ln_g,
           ln_b, W_v2, b_v2):
    f32 = jnp.float32

    feats_p = jnp.zeros((NPAD, DPAD), f32).at[:N, :DF].set(feats153)
    Wa = jnp.zeros((DPAD, H), f32).at[:DF].set(W_e[:DF] - W_e[DF:])
    Wc = jnp.zeros((DPAD, H), f32).at[:DF].set(W_e[DF:])

    sq = jnp.sum(support_xyz * support_xyz, axis=1)
    bi_f = batch_index.astype(f32)
    pcore = jnp.concatenate(
        [support_xyz, sq[:, None], bi_f[:, None], jnp.zeros((N, 3), f32)],
        axis=1)
    prow_pad = jnp.zeros((NPAD - N, 8), f32).at[:, 4].set(-1.0)
    P = jnp.concatenate([pcore, prow_pad], axis=0)
    PT3 = jnp.transpose(P.T.reshape(8, NCH, CCH), (1, 0, 2))

    ar = jnp.arange(B, dtype=batch_index.dtype)
    starts = jnp.searchsorted(batch_index, ar, side='left').astype(jnp.int32)
    ends = jnp.searchsorted(batch_index, ar, side='right').astype(jnp.int32)
    first = jnp.minimum(jnp.arange(NBLK) * R, N - 1)
    last = jnp.minimum(jnp.arange(NBLK) * R + R - 1, N - 1)
    c_lo = starts[batch_index[first]]
    c_hi = ends[batch_index[last]]
    kc0 = c_lo // CCH
    kc1 = (c_hi + CCH - 1) // CCH
    bounds = jnp.stack([kc0, kc1], axis=1).astype(jnp.int32)

    lang_emb = pl.pallas_call(
        _lang_body,
        out_shape=jax.ShapeDtypeStruct((B, H), f32),
    )(lang_rel_feats, W_l1, b_l1, bn_g, bn_b, W_l2, b_l2)

    a, c = pl.pallas_call(
        _pre_body,
        grid=(NBLK,),
        in_specs=[
            pl.BlockSpec((R, DPAD), lambda i: (i, 0)),
            pl.BlockSpec((DPAD, H), lambda i: (0, 0)),
            pl.BlockSpec((DPAD, H), lambda i: (0, 0)),
            pl.BlockSpec((H,), lambda i: (0,)),
        ],
        out_specs=[
            pl.BlockSpec((R, H), lambda i: (i, 0)),
            pl.BlockSpec((R, H), lambda i: (i, 0)),
        ],
        out_shape=[jax.ShapeDtypeStruct((NPAD, H), f32)] * 2,
    )(feats_p, Wa, Wc, b_e)

    idx = pl.pallas_call(
        _knn_body,
        grid=(NBLK,),
        in_specs=[
            pl.BlockSpec(memory_space=pltpu.SMEM),
            pl.BlockSpec((R, 8), lambda i: (i, 0)),
            pl.BlockSpec((NCH, 8, CCH), lambda i: (0, 0, 0)),
        ],
        out_specs=pl.BlockSpec((R, K), lambda i: (i, 0)),
        out_shape=jax.ShapeDtypeStruct((NPAD, K), jnp.int32),
    )(bounds, P, PT3)

    g = _sc_gather_max(idx.reshape(NPAD * K), c)

    scores = pl.pallas_call(
        _vis_body,
        grid=(NBLK,),
        in_specs=[
            pl.BlockSpec((R, H), lambda i: (i, 0)),
            pl.BlockSpec((R, H), lambda i: (i, 0)),
            pl.BlockSpec((R, 8), lambda i: (i, 0)),
            pl.BlockSpec((B, H), lambda i: (0, 0)),
            pl.BlockSpec((H, H), lambda i: (0, 0)),
            pl.BlockSpec((H,), lambda i: (0,)),
            pl.BlockSpec((H,), lambda i: (0,)),
            pl.BlockSpec((H,), lambda i: (0,)),
            pl.BlockSpec((H, H), lambda i: (0, 0)),
            pl.BlockSpec((H,), lambda i: (0,)),
        ],
        out_specs=pl.BlockSpec((R, 1), lambda i: (i, 0)),
        out_shape=jax.ShapeDtypeStruct((NPAD, 1), f32),
    )(a, g, P, lang_emb, W_v1, b_v1, ln_g, ln_b, W_v2, b_v2)

    return scores[:N, 0]


# speedup vs baseline: 17.8898x; 17.8898x over previous
import functools

import jax
import jax.numpy as jnp
from jax import lax
from jax.experimental import pallas as pl
from jax.experimental.pallas import tpu as pltpu
from jax.experimental.pallas import tpu_sc as plsc

N = 10000
B = 32
K = 16
DF = 153
DL = 256
H = 128

NPAD = 10240
R = 256
CCH = 512
NBLK = NPAD // R
NCH = NPAD // CCH
DPAD = 256

BIG = 1e9
FINF = 3e38

NC = 2
NS = 16
NW = NC * NS
NODES_PW = NPAD // NW
GCH = 8
NGB = 4
STEP = GCH * NGB
OUTER = NODES_PW // STEP


def _lang_body(lang_ref, wl1_ref, bl1_ref, bng_ref, bnb_ref, wl2_ref, bl2_ref,
               out_ref):
    h = jnp.dot(lang_ref[...], wl1_ref[...],
                preferred_element_type=jnp.float32) + bl1_ref[...][None, :]
    mu = jnp.mean(h, axis=0, keepdims=True)
    var = jnp.mean((h - mu) ** 2, axis=0, keepdims=True)
    h = (h - mu) / jnp.sqrt(var + 1e-5) * bng_ref[...][None, :] \
        + bnb_ref[...][None, :]
    h = jnp.maximum(h, 0.0)
    out_ref[...] = jnp.dot(h, wl2_ref[...],
                           preferred_element_type=jnp.float32) \
        + bl2_ref[...][None, :]


def _pre_body(f_ref, wa_ref, wc_ref, be_ref, a_ref, c_ref):
    f = f_ref[...]
    a_ref[...] = jnp.dot(f, wa_ref[...],
                         preferred_element_type=jnp.float32) \
        + be_ref[...][None, :]
    c_ref[...] = jnp.dot(f, wc_ref[...], preferred_element_type=jnp.float32)


def _knn_body(bounds_ref, p_ref, pt3_ref, out_ref):
    blk = pl.program_id(0)
    r0 = blk * R
    row = p_ref[...]
    xr = row[:, 0:1]
    yr = row[:, 1:2]
    zr = row[:, 2:3]
    sqr = row[:, 3:4]
    bir = row[:, 4:5]
    rowid = r0 + lax.broadcasted_iota(jnp.int32, (R, 1), 0)
    kc0 = bounds_ref[blk, 0]
    kc1 = bounds_ref[blk, 1]

    init_v = jnp.full((R, K), FINF, jnp.float32)
    init_i = jnp.broadcast_to(
        lax.broadcasted_iota(jnp.int32, (1, K), 1) + NPAD, (R, K))

    def chunk_body(kc, carry):
        cv, ci = carry
        c0 = kc * CCH
        ch = pt3_ref[kc]
        xc = ch[0:1, :]
        yc = ch[1:2, :]
        zc = ch[2:3, :]
        sqc = ch[3:4, :]
        bic = ch[4:5, :]
        dot = xr * xc + yr * yc + zr * zc
        d2 = sqr + sqc - 2.0 * dot
        colid = c0 + lax.broadcasted_iota(jnp.int32, (1, CCH), 1)
        d2 = d2 + jnp.where(bir != bic, BIG, 0.0) \
            + jnp.where(rowid == colid, BIG, 0.0)
        cand_v = jnp.concatenate([cv, d2], axis=1)
        cand_i = jnp.concatenate(
            [ci, jnp.broadcast_to(colid, (R, CCH))], axis=1)
        vs = []
        isel = []
        for _ in range(K):
            m = jnp.min(cand_v, axis=1, keepdims=True)
            si = jnp.min(jnp.where(cand_v == m, cand_i, jnp.int32(2**31 - 1)),
                         axis=1, keepdims=True)
            vs.append(m)
            isel.append(si)
            cand_v = jnp.where(cand_i == si, FINF, cand_v)
        return jnp.concatenate(vs, axis=1), jnp.concatenate(isel, axis=1)

    _, idx = lax.fori_loop(kc0, kc1, chunk_body, (init_v, init_i))
    out_ref[...] = jnp.clip(idx, 0, NPAD - 1)


def _vis_body(a_ref, g_ref, p_ref, lang_ref, wv1_ref, bv1_ref, lng_ref,
              lnb_ref, wv2_ref, bv2_ref, out_ref):
    node = jnp.maximum(a_ref[...] + g_ref[...], 0.0)
    v = jnp.dot(node, wv1_ref[...],
                preferred_element_type=jnp.float32) + bv1_ref[...][None, :]
    mu = jnp.mean(v, axis=1, keepdims=True)
    var = jnp.mean((v - mu) ** 2, axis=1, keepdims=True)
    v = (v - mu) / jnp.sqrt(var + 1e-5) * lng_ref[...][None, :] \
        + lnb_ref[...][None, :]
    v = jnp.maximum(v, 0.0)
    v = jnp.dot(v, wv2_ref[...],
                preferred_element_type=jnp.float32) + bv2_ref[...][None, :]
    bi = p_ref[...][:, 4:5]
    ioti = lax.broadcasted_iota(jnp.int32, (R, B), 1).astype(jnp.float32)
    onehot = (bi == ioti).astype(jnp.float32)
    lf = jnp.dot(onehot, lang_ref[...], preferred_element_type=jnp.float32)
    num = jnp.sum(v * lf, axis=1, keepdims=True)
    den = jnp.sqrt(jnp.sum(v * v, axis=1, keepdims=True)) \
        * jnp.sqrt(jnp.sum(lf * lf, axis=1, keepdims=True)) + 1e-8
    out_ref[...] = num / den


def _sc_gather_max(idx_flat, ctab):
    mesh = plsc.VectorSubcoreMesh(core_axis_name="c", subcore_axis_name="s",
                                  num_cores=NC, num_subcores=NS)

    @functools.partial(
        pl.kernel,
        out_type=jax.ShapeDtypeStruct((NPAD, H), jnp.float32),
        mesh=mesh,
        scratch_types=(
            [pltpu.VMEM((GCH * K,), jnp.int32) for _ in range(NGB)]
            + [pltpu.VMEM((GCH * K, H), jnp.float32) for _ in range(NGB)]
            + [pltpu.VMEM((STEP, H), jnp.float32), pltpu.SemaphoreType.DMA]
        ),
    )
    def kfn(idx_hbm, c_hbm, out_hbm, i0, i1, i2, i3, r0, r1, r2, r3, outv,
            sem):
        ivs = (i0, i1, i2, i3)
        rvs = (r0, r1, r2, r3)
        wid = lax.axis_index("s") * NC + lax.axis_index("c")
        base = wid * NODES_PW

        def outer(t, carry):
            node0 = base + t * STEP
            for j in range(NGB):
                pltpu.sync_copy(
                    idx_hbm.at[pl.ds((node0 + j * GCH) * K, GCH * K)], ivs[j])
            cps = [pltpu.async_copy(c_hbm.at[ivs[j]], rvs[j], sem)
                   for j in range(NGB)]
            for cp in cps:
                cp.wait()
            for j in range(NGB):
                rv = rvs[j]

                def nbody(n, c2, rv=rv, j=j):
                    rbase = n * K
                    for dg in range(H // 16):
                        acc = rv[rbase, pl.ds(dg * 16, 16)]
                        for k in range(1, K):
                            acc = jnp.maximum(
                                acc, rv[rbase + k, pl.ds(dg * 16, 16)])
                        outv[j * GCH + n, pl.ds(dg * 16, 16)] = acc
                    return c2

                lax.fori_loop(0, GCH, nbody, 0)
            pltpu.sync_copy(outv, out_hbm.at[pl.ds(node0, STEP)])
            return carry

        lax.fori_loop(0, OUTER, outer, 0)

    return kfn(idx_flat, ctab)


def kernel(feats153, support_xyz, batch_index, filtered_index, lang_rel_feats,
           W_l1, b_l1, bn_g, bn_b, W_l2, b_l2, W_e, b_e, W_v1, b_v1, ln_g,
           ln_b, W_v2, b_v2):
    f32 = jnp.float32

    feats_p = jnp.zeros((NPAD, DPAD), f32).at[:N, :DF].set(feats153)
    Wa = jnp.zeros((DPAD, H), f32).at[:DF].set(W_e[:DF] - W_e[DF:])
    Wc = jnp.zeros((DPAD, H), f32).at[:DF].set(W_e[DF:])

    sq = jnp.sum(support_xyz * support_xyz, axis=1)
    u = lax.bitcast_convert_type(support_xyz, jnp.uint32)
    u = (u + jnp.uint32(0x7FFF) + ((u >> 16) & jnp.uint32(1))) \
        & jnp.uint32(0xFFFF0000)
    xyz_r = lax.bitcast_convert_type(u, jnp.float32)
    bi_f = batch_index.astype(f32)
    pcore = jnp.concatenate(
        [xyz_r, sq[:, None], bi_f[:, None], jnp.zeros((N, 3), f32)],
        axis=1)
    prow_pad = jnp.zeros((NPAD - N, 8), f32).at[:, 4].set(-1.0)
    P = jnp.concatenate([pcore, prow_pad], axis=0)
    PT3 = jnp.transpose(P.T.reshape(8, NCH, CCH), (1, 0, 2))

    ar = jnp.arange(B, dtype=batch_index.dtype)
    starts = jnp.searchsorted(batch_index, ar, side='left').astype(jnp.int32)
    ends = jnp.searchsorted(batch_index, ar, side='right').astype(jnp.int32)
    first = jnp.minimum(jnp.arange(NBLK) * R, N - 1)
    last = jnp.minimum(jnp.arange(NBLK) * R + R - 1, N - 1)
    c_lo = starts[batch_index[first]]
    c_hi = ends[batch_index[last]]
    kc0 = c_lo // CCH
    kc1 = (c_hi + CCH - 1) // CCH
    bounds = jnp.stack([kc0, kc1], axis=1).astype(jnp.int32)

    lang_emb = pl.pallas_call(
        _lang_body,
        out_shape=jax.ShapeDtypeStruct((B, H), f32),
    )(lang_rel_feats, W_l1, b_l1, bn_g, bn_b, W_l2, b_l2)

    a, c = pl.pallas_call(
        _pre_body,
        grid=(NBLK,),
        in_specs=[
            pl.BlockSpec((R, DPAD), lambda i: (i, 0)),
            pl.BlockSpec((DPAD, H), lambda i: (0, 0)),
            pl.BlockSpec((DPAD, H), lambda i: (0, 0)),
            pl.BlockSpec((H,), lambda i: (0,)),
        ],
        out_specs=[
            pl.BlockSpec((R, H), lambda i: (i, 0)),
            pl.BlockSpec((R, H), lambda i: (i, 0)),
        ],
        out_shape=[jax.ShapeDtypeStruct((NPAD, H), f32)] * 2,
    )(feats_p, Wa, Wc, b_e)

    idx = pl.pallas_call(
        _knn_body,
        grid=(NBLK,),
        in_specs=[
            pl.BlockSpec(memory_space=pltpu.SMEM),
            pl.BlockSpec((R, 8), lambda i: (i, 0)),
            pl.BlockSpec((NCH, 8, CCH), lambda i: (0, 0, 0)),
        ],
        out_specs=pl.BlockSpec((R, K), lambda i: (i, 0)),
        out_shape=jax.ShapeDtypeStruct((NPAD, K), jnp.int32),
    )(bounds, P, PT3)

    g = _sc_gather_max(idx.reshape(NPAD * K), c)

    scores = pl.pallas_call(
        _vis_body,
        grid=(NBLK,),
        in_specs=[
            pl.BlockSpec((R, H), lambda i: (i, 0)),
            pl.BlockSpec((R, H), lambda i: (i, 0)),
            pl.BlockSpec((R, 8), lambda i: (i, 0)),
            pl.BlockSpec((B, H), lambda i: (0, 0)),
            pl.BlockSpec((H, H), lambda i: (0, 0)),
            pl.BlockSpec((H,), lambda i: (0,)),
            pl.BlockSpec((H,), lambda i: (0,)),
            pl.BlockSpec((H,), lambda i: (0,)),
            pl.BlockSpec((H, H), lambda i: (0, 0)),
            pl.BlockSpec((H,), lambda i: (0,)),
        ],
        out_specs=pl.BlockSpec((R, 1), lambda i: (i, 0)),
        out_shape=jax.ShapeDtypeStruct((NPAD, 1), f32),
    )(a, g, P, lang_emb, W_v1, b_v1, ln_g, ln_b, W_v2, b_v2)

    return scores[:N, 0]
